# baseline (speedup 1.0000x reference)
import functools

import jax
import jax.numpy as jnp
import numpy as np
from jax import lax
from jax.experimental import pallas as pl
from jax.experimental.pallas import tpu as pltpu
from jax.experimental.pallas import tpu_sc as plsc

_N = 10000
_D = 128
_K = 96
_NSUB = 16
_BLK = 16
_PAD = _NSUB * 8 * _K
_RPT = 624


def _mm_in_body(x_ref, w_ref, o_ref):
    o_ref[...] = jnp.dot(x_ref[...], w_ref[...],
                         preferred_element_type=jnp.float32)


def _dense_in(x, wt):
    return pl.pallas_call(
        _mm_in_body,
        grid=(10,),
        in_specs=[pl.BlockSpec((_N // 10, _D), lambda i: (i, 0)),
                  pl.BlockSpec((_D, _D), lambda i: (0, 0))],
        out_specs=pl.BlockSpec((_N // 10, _D), lambda i: (i, 0)),
        out_shape=jax.ShapeDtypeStruct((_N, _D), jnp.float32),
    )(x, wt)


def _proj_body(h0_ref, h1_ref, h2_ref, h3_ref, h4_ref, wt_ref, b_ref, o_ref):
    acc = jnp.dot(h0_ref[...], wt_ref[0:128, :],
                  preferred_element_type=jnp.float32)
    acc = acc + jnp.dot(h1_ref[...], wt_ref[128:256, :],
                        preferred_element_type=jnp.float32)
    acc = acc + jnp.dot(h2_ref[...], wt_ref[256:384, :],
                        preferred_element_type=jnp.float32)
    acc = acc + jnp.dot(h3_ref[...], wt_ref[384:512, :],
                        preferred_element_type=jnp.float32)
    acc = acc + jnp.dot(h4_ref[...], wt_ref[512:640, :],
                        preferred_element_type=jnp.float32)
    o_ref[...] = acc + b_ref[...]


def _proj(h0, h1, h2, h3, h4, wt, b2):
    blk = _N // 10
    hspec = pl.BlockSpec((blk, _D), lambda i: (i, 0))
    return pl.pallas_call(
        _proj_body,
        grid=(10,),
        in_specs=[hspec] * 5 +
                 [pl.BlockSpec((640, 64), lambda i: (0, 0)),
                  pl.BlockSpec((1, 64), lambda i: (0, 0))],
        out_specs=pl.BlockSpec((blk, 64), lambda i: (i, 0)),
        out_shape=jax.ShapeDtypeStruct((_N, 64), jnp.float32),
    )(h0, h1, h2, h3, h4, wt, b2)


def _lane_splat(v16, el):
    lane = (jnp.zeros((16,), jnp.int32) + el)[:, None]
    return lax.gather(
        v16, lane,
        lax.GatherDimensionNumbers(offset_dims=(), collapsed_slice_dims=(0,),
                                   start_index_map=(0,)),
        slice_sizes=(1,),
        mode=lax.GatherScatterMode.PROMISE_IN_BOUNDS)


def _sc_pair_body(ha, hb, ra, ca, va, rb, cb, vb, out_a, out_b,
                  colb, rowb, valb, g0, g1, o0, o1, acc,
                  sg0, sg1, sh0, sh1, ss0, ss1):
    cid = lax.axis_index("c")
    sid = lax.axis_index("s")
    gbufs = (g0, g1)
    obufs = (o0, o1)
    gsems = (sg0, sg1)
    gsems2 = (sh0, sh1)
    ssems = (ss0, ss1)

    def _zrow(r, carry):
        for j in range(8):
            o0[r, pl.ds(16 * j, 16)] = jnp.zeros((16,), jnp.float32)
        return carry
    lax.fori_loop(0, _K, _zrow, 0)
    for t in range(6):
        pltpu.sync_copy(o0.at[pl.ds(0, 96)],
                        acc.at[pl.ds(sid * _RPT + t * 96, 96)])
    pltpu.sync_copy(o0.at[pl.ds(0, 48)],
                    acc.at[pl.ds(sid * _RPT + 576, 48)])

    @pl.when(sid == _NSUB - 1)
    def _():
        pltpu.sync_copy(o0.at[pl.ds(0, 16)],
                        acc.at[pl.ds(_NSUB * _RPT, 16)])

    def _run(h, rows2d, cols2d, vals2d, out_h):
        nch = cols2d.shape[0] // _NSUB
        cbase = sid * nch

        def _parity(c):
            return (c // _BLK) % 2

        def _slot(c):
            return c % _BLK

        def _load_block(c0):
            p = _parity(c0)
            b8 = pl.multiple_of(cbase + c0, 8)
            pltpu.sync_copy(cols2d.at[pl.ds(b8, _BLK)], colb.at[p])
            pltpu.sync_copy(rows2d.at[pl.ds(b8, _BLK)], rowb.at[p])
            pltpu.sync_copy(vals2d.at[pl.ds(b8, _BLK)], valb.at[p])

        _H = _K // 2

        def _start_gather(c, k):
            p, j = _parity(c), _slot(c)
            pltpu.async_copy(h.at[colb.at[p, j, pl.ds(0, _H)]],
                             gbufs[k].at[pl.ds(0, _H)], gsems[k])
            pltpu.async_copy(h.at[colb.at[p, j, pl.ds(_H, _H)]],
                             gbufs[k].at[pl.ds(_H, _H)], gsems2[k])

        def _wait_gather(c, k):
            p, j = _parity(c), _slot(c)
            pltpu.make_async_copy(h.at[colb.at[p, j, pl.ds(0, _H)]],
                                  gbufs[k].at[pl.ds(0, _H)],
                                  gsems[k]).wait()
            pltpu.make_async_copy(h.at[colb.at[p, j, pl.ds(_H, _H)]],
                                  gbufs[k].at[pl.ds(_H, _H)],
                                  gsems2[k]).wait()

        def _start_scatter(c, k):
            pltpu.async_copy(obufs[k],
                             acc.at[rowb.at[_parity(c), _slot(c)]],
                             ssems[k], add=True)

        def _wait_scatter(c, k):
            pltpu.make_async_copy(obufs[k],
                                  acc.at[rowb.at[_parity(c), _slot(c)]],
                                  ssems[k]).wait()

        def _step(ci, k):
            @pl.when(ci >= 2)
            def _():
                _wait_scatter(ci - 2, k)

            @pl.when(jnp.logical_and((ci + 1) % _BLK == 0, ci + 1 < nch))
            def _():
                _load_block(ci + 1)

            @pl.when(ci + 1 < nch)
            def _():
                _start_gather(ci + 1, 1 - k)

            _wait_gather(ci, k)

            p = _parity(ci)
            j = _slot(ci)
            gb = gbufs[k]
            ob = obufs[k]

            def _group(g, carry):
                v16 = valb[p, j, pl.ds(16 * g, 16)]
                for el in range(16):
                    e = 16 * g + el
                    vsp = _lane_splat(v16, el)
                    for q in range(4):
                        ev, od = plsc.unpack(
                            gb[e, pl.ds(32 * q, 32)],
                            format=plsc.PackFormat.INTERLEAVED,
                            preferred_element_type=jnp.float32)
                        ob[e, pl.ds(32 * q, 16)] = ev * vsp
                        ob[e, pl.ds(32 * q + 16, 16)] = od * vsp
                return carry
            lax.fori_loop(0, _K // 16, _group, 0)

            _start_scatter(ci, k)

        _load_block(0)
        _start_gather(0, 0)
        plsc.subcore_barrier()

        def _pair(ti, carry):
            _step(2 * ti, 0)
            _step(2 * ti + 1, 1)
            return carry
        lax.fori_loop(0, nch // 2, _pair, 0)

        _wait_scatter(jnp.int32(nch - 2), 0)
        _wait_scatter(jnp.int32(nch - 1), 1)
        plsc.subcore_barrier()

        pltpu.sync_copy(acc.at[pl.ds(sid * _RPT, _RPT)],
                        out_h.at[pl.ds(sid * _RPT, _RPT)])

        @pl.when(sid == _NSUB - 1)
        def _():
            pltpu.sync_copy(acc.at[pl.ds(_NSUB * _RPT, 16)],
                            out_h.at[pl.ds(_NSUB * _RPT, 16)])

    @pl.when(cid == 0)
    def _():
        _run(ha, ra, ca, va, out_a)

    @pl.when(cid == 1)
    def _():
        _run(hb, rb, cb, vb, out_b)


_spmm_pair = functools.partial(
    pl.kernel,
    mesh=plsc.VectorSubcoreMesh(core_axis_name="c", subcore_axis_name="s"),
    compiler_params=pltpu.CompilerParams(use_tc_tiling_on_sc=False,
                                         needs_layout_passes=False),
    out_type=(jax.ShapeDtypeStruct((_N, _D), jnp.float32),
              jax.ShapeDtypeStruct((_N, _D), jnp.float32)),
    scratch_types=[
        pltpu.VMEM((2, _BLK, _K), jnp.int32),
        pltpu.VMEM((2, _BLK, _K), jnp.int32),
        pltpu.VMEM((2, _BLK, _K), jnp.float32),
        pltpu.VMEM((_K, _D), jnp.bfloat16),
        pltpu.VMEM((_K, _D), jnp.bfloat16),
        pltpu.VMEM((_K, _D), jnp.float32),
        pltpu.VMEM((_K, _D), jnp.float32),
        pltpu.VMEM_SHARED((_N, _D), jnp.float32),
        pltpu.SemaphoreType.DMA,
        pltpu.SemaphoreType.DMA,
        pltpu.SemaphoreType.DMA,
        pltpu.SemaphoreType.DMA,
        pltpu.SemaphoreType.DMA,
        pltpu.SemaphoreType.DMA,
    ],
)(_sc_pair_body)


def _prep_edges(rows, cols, val):
    e = val.shape[0]
    ep = -(-e // _PAD) * _PAD
    pad = ep - e
    return (jnp.pad(rows, (0, pad)).reshape(-1, _K),
            jnp.pad(cols, (0, pad)).reshape(-1, _K),
            jnp.pad(val, (0, pad)).reshape(-1, _K))


_PERM = np.concatenate(
    [32 * b + np.concatenate([np.arange(0, 32, 2), np.arange(1, 32, 2)])
     for b in range(4)])
_PERM2 = _PERM[_PERM]


def kernel(x, adj1_indices, adj1_values, adj2_indices, adj2_values,
           W1, W_out, b_out):
    a1 = _prep_edges(adj1_indices[0], adj1_indices[1], adj1_values)
    a2 = _prep_edges(adj2_indices[0], adj2_indices[1], adj2_values)
    h0 = _dense_in(x, W1.T)
    h0b = h0.astype(jnp.bfloat16)
    h1, h2 = _spmm_pair(h0b, h0b, *a1, *a2)
    h1b = h1.astype(jnp.bfloat16)
    h3, h4 = _spmm_pair(h1b, h1b, *a1, *a2)
    wt = W_out.T
    wtp = jnp.concatenate([
        wt[0:128],
        wt[128:256][_PERM], wt[256:384][_PERM],
        wt[384:512][_PERM2], wt[512:640][_PERM2]], axis=0)
    return _proj(h0, h1, h2, h3, h4, wtp, b_out.reshape(1, 64))


# speedup vs baseline: 1.0029x; 1.0029x over previous
import functools

import jax
import jax.numpy as jnp
import numpy as np
from jax import lax
from jax.experimental import pallas as pl
from jax.experimental.pallas import tpu as pltpu
from jax.experimental.pallas import tpu_sc as plsc

_N = 10000
_D = 128
_K = 96
_NSUB = 16
_BLK = 16
_PAD = _NSUB * 8 * _K
_RPT = 624


def _mm_in_body(x_ref, w_ref, o_ref):
    o_ref[...] = jnp.dot(x_ref[...], w_ref[...],
                         preferred_element_type=jnp.float32)


def _dense_in(x, wt):
    return pl.pallas_call(
        _mm_in_body,
        grid=(10,),
        in_specs=[pl.BlockSpec((_N // 10, _D), lambda i: (i, 0)),
                  pl.BlockSpec((_D, _D), lambda i: (0, 0))],
        out_specs=pl.BlockSpec((_N // 10, _D), lambda i: (i, 0)),
        out_shape=jax.ShapeDtypeStruct((_N, _D), jnp.float32),
    )(x, wt)


def _proj_body(h0_ref, h1_ref, h2_ref, h3_ref, h4_ref, wt_ref, b_ref, o_ref):
    acc = jnp.dot(h0_ref[...], wt_ref[0:128, :],
                  preferred_element_type=jnp.float32)
    acc = acc + jnp.dot(h1_ref[...], wt_ref[128:256, :],
                        preferred_element_type=jnp.float32)
    acc = acc + jnp.dot(h2_ref[...], wt_ref[256:384, :],
                        preferred_element_type=jnp.float32)
    acc = acc + jnp.dot(h3_ref[...], wt_ref[384:512, :],
                        preferred_element_type=jnp.float32)
    acc = acc + jnp.dot(h4_ref[...], wt_ref[512:640, :],
                        preferred_element_type=jnp.float32)
    o_ref[...] = acc + b_ref[...]


def _proj(h0, h1, h2, h3, h4, wt, b2):
    blk = _N // 10
    hspec = pl.BlockSpec((blk, _D), lambda i: (i, 0))
    return pl.pallas_call(
        _proj_body,
        grid=(10,),
        in_specs=[hspec] * 5 +
                 [pl.BlockSpec((640, 64), lambda i: (0, 0)),
                  pl.BlockSpec((1, 64), lambda i: (0, 0))],
        out_specs=pl.BlockSpec((blk, 64), lambda i: (i, 0)),
        out_shape=jax.ShapeDtypeStruct((_N, 64), jnp.float32),
    )(h0, h1, h2, h3, h4, wt, b2)


def _lane_splat(v16, el):
    lane = (jnp.zeros((16,), jnp.int32) + el)[:, None]
    return lax.gather(
        v16, lane,
        lax.GatherDimensionNumbers(offset_dims=(), collapsed_slice_dims=(0,),
                                   start_index_map=(0,)),
        slice_sizes=(1,),
        mode=lax.GatherScatterMode.PROMISE_IN_BOUNDS)


def _sc_pair_body(ha, hb, ra, ca, va, rb, cb, vb, out_a, out_b,
                  colb, rowb, valb, g0, g1, o0, o1, acc,
                  sg0, sg1, ss0, ss1):
    cid = lax.axis_index("c")
    sid = lax.axis_index("s")
    gbufs = (g0, g1)
    obufs = (o0, o1)
    gsems = (sg0, sg1)
    ssems = (ss0, ss1)

    def _zrow(r, carry):
        for j in range(8):
            o0[r, pl.ds(16 * j, 16)] = jnp.zeros((16,), jnp.float32)
        return carry
    lax.fori_loop(0, _K, _zrow, 0)
    for t in range(6):
        pltpu.sync_copy(o0.at[pl.ds(0, 96)],
                        acc.at[pl.ds(sid * _RPT + t * 96, 96)])
    pltpu.sync_copy(o0.at[pl.ds(0, 48)],
                    acc.at[pl.ds(sid * _RPT + 576, 48)])

    @pl.when(sid == _NSUB - 1)
    def _():
        pltpu.sync_copy(o0.at[pl.ds(0, 16)],
                        acc.at[pl.ds(_NSUB * _RPT, 16)])

    def _run(h, rows2d, cols2d, vals2d, out_h):
        nch = cols2d.shape[0] // _NSUB
        cbase = sid * nch

        def _parity(c):
            return (c // _BLK) % 2

        def _slot(c):
            return c % _BLK

        def _load_block(c0):
            p = _parity(c0)
            b8 = pl.multiple_of(cbase + c0, 8)
            pltpu.sync_copy(cols2d.at[pl.ds(b8, _BLK)], colb.at[p])
            pltpu.sync_copy(rows2d.at[pl.ds(b8, _BLK)], rowb.at[p])
            pltpu.sync_copy(vals2d.at[pl.ds(b8, _BLK)], valb.at[p])

        def _start_gather(c, k):
            pltpu.async_copy(h.at[colb.at[_parity(c), _slot(c)]],
                             gbufs[k], gsems[k])

        def _wait_gather(c, k):
            pltpu.make_async_copy(h.at[colb.at[_parity(c), _slot(c)]],
                                  gbufs[k], gsems[k]).wait()

        def _start_scatter(c, k):
            pltpu.async_copy(obufs[k],
                             acc.at[rowb.at[_parity(c), _slot(c)]],
                             ssems[k], add=True)

        def _wait_scatter(c, k):
            pltpu.make_async_copy(obufs[k],
                                  acc.at[rowb.at[_parity(c), _slot(c)]],
                                  ssems[k]).wait()

        def _step(ci, k):
            @pl.when(ci >= 2)
            def _():
                _wait_scatter(ci - 2, k)

            @pl.when(jnp.logical_and((ci + 1) % _BLK == 0, ci + 1 < nch))
            def _():
                _load_block(ci + 1)

            @pl.when(ci + 1 < nch)
            def _():
                _start_gather(ci + 1, 1 - k)

            _wait_gather(ci, k)

            p = _parity(ci)
            j = _slot(ci)
            gb = gbufs[k]
            ob = obufs[k]

            def _group(g, carry):
                v16 = valb[p, j, pl.ds(16 * g, 16)]
                for el in range(16):
                    e = 16 * g + el
                    vsp = _lane_splat(v16, el)
                    for q in range(4):
                        ev, od = plsc.unpack(
                            gb[e, pl.ds(32 * q, 32)],
                            format=plsc.PackFormat.INTERLEAVED,
                            preferred_element_type=jnp.float32)
                        ob[e, pl.ds(32 * q, 16)] = ev * vsp
                        ob[e, pl.ds(32 * q + 16, 16)] = od * vsp
                return carry
            lax.fori_loop(0, _K // 16, _group, 0)

            _start_scatter(ci, k)

        _load_block(0)
        _start_gather(0, 0)
        plsc.subcore_barrier()

        def _pair(ti, carry):
            _step(2 * ti, 0)
            _step(2 * ti + 1, 1)
            return carry
        lax.fori_loop(0, nch // 2, _pair, 0)

        _wait_scatter(jnp.int32(nch - 2), 0)
        _wait_scatter(jnp.int32(nch - 1), 1)
        plsc.subcore_barrier()

        pltpu.sync_copy(acc.at[pl.ds(sid * _RPT, _RPT)],
                        out_h.at[pl.ds(sid * _RPT, _RPT)])

        @pl.when(sid == _NSUB - 1)
        def _():
            pltpu.sync_copy(acc.at[pl.ds(_NSUB * _RPT, 16)],
                            out_h.at[pl.ds(_NSUB * _RPT, 16)])

    @pl.when(cid == 0)
    def _():
        _run(ha, ra, ca, va, out_a)

    @pl.when(cid == 1)
    def _():
        _run(hb, rb, cb, vb, out_b)


_spmm_pair = functools.partial(
    pl.kernel,
    mesh=plsc.VectorSubcoreMesh(core_axis_name="c", subcore_axis_name="s"),
    compiler_params=pltpu.CompilerParams(use_tc_tiling_on_sc=False,
                                         needs_layout_passes=False),
    out_type=(jax.ShapeDtypeStruct((_N, _D), jnp.float32),
              jax.ShapeDtypeStruct((_N, _D), jnp.float32)),
    scratch_types=[
        pltpu.VMEM((2, _BLK, _K), jnp.int32),
        pltpu.VMEM((2, _BLK, _K), jnp.int32),
        pltpu.VMEM((2, _BLK, _K), jnp.float32),
        pltpu.VMEM((_K, _D), jnp.bfloat16),
        pltpu.VMEM((_K, _D), jnp.bfloat16),
        pltpu.VMEM((_K, _D), jnp.float32),
        pltpu.VMEM((_K, _D), jnp.float32),
        pltpu.VMEM_SHARED((_N, _D), jnp.float32),
        pltpu.SemaphoreType.DMA,
        pltpu.SemaphoreType.DMA,
        pltpu.SemaphoreType.DMA,
        pltpu.SemaphoreType.DMA,
    ],
)(_sc_pair_body)


def _prep_edges(rows, cols, val):
    e = val.shape[0]
    ep = -(-e // _PAD) * _PAD
    pad = ep - e
    return (jnp.pad(rows, (0, pad)).reshape(-1, _K),
            jnp.pad(cols, (0, pad)).reshape(-1, _K),
            jnp.pad(val, (0, pad)).reshape(-1, _K))


_PERM = np.concatenate(
    [32 * b + np.concatenate([np.arange(0, 32, 2), np.arange(1, 32, 2)])
     for b in range(4)])
_PERM2 = _PERM[_PERM]


def kernel(x, adj1_indices, adj1_values, adj2_indices, adj2_values,
           W1, W_out, b_out):
    a1 = _prep_edges(adj1_indices[0], adj1_indices[1], adj1_values)
    a2 = _prep_edges(adj2_indices[0], adj2_indices[1], adj2_values)
    h0 = _dense_in(x, W1.T)
    h0b = h0.astype(jnp.bfloat16)
    h1, h2 = _spmm_pair(h0b, h0b, *a1, *a2)
    h1b = h1.astype(jnp.bfloat16)
    h3, h4 = _spmm_pair(h1b, h1b, *a1, *a2)
    wt = W_out.T
    wtp = jnp.concatenate([
        wt[0:128],
        wt[128:256][_PERM], wt[256:384][_PERM],
        wt[384:512][_PERM2], wt[512:640][_PERM2]], axis=0)
    return _proj(h0, h1, h2, h3, h4, wtp, b_out.reshape(1, 64))
